# baseline (speedup 1.0000x reference)
import jax
import jax.numpy as jnp
from jax import lax
from jax.experimental import pallas as pl
from jax.experimental.pallas import tpu as pltpu
from jax.experimental.pallas import tpu_sc as plsc

_N = 100
_PAD = 128
_NUM_IDX = 25


def _rotl(x, r):
    return (x << jnp.uint32(r)) | (x >> jnp.uint32(32 - r))


def _threefry2x32(k0, k1, x0, x1):
    ks = [k0, k1, k0 ^ k1 ^ jnp.uint32(0x1BD11BDA)]
    x0 = x0 + ks[0]
    x1 = x1 + ks[1]
    rotations = ((13, 15, 26, 6), (17, 29, 16, 24))
    for i in range(5):
        for r in rotations[i % 2]:
            x0 = x0 + x1
            x1 = _rotl(x1, r)
            x1 = x0 ^ x1
        x0 = x0 + ks[(i + 1) % 3]
        x1 = x1 + ks[(i + 2) % 3] + jnp.uint32(i + 1)
    return x0, x1


def _mask_body(out_ref, mask_ref):
    zero = jnp.uint32(0)
    k0, k1 = _threefry2x32(zero, zero, zero, jnp.uint32(1))

    for j in range(_PAD):
        mask_ref[j] = jnp.int32(0)

    for e in range(_NUM_IDX):
        b0, b1 = _threefry2x32(k0, k1, zero, jnp.uint32(e))
        bits = b0 ^ b1
        f = lax.bitcast_convert_type(
            (bits >> jnp.uint32(9)) | jnp.uint32(0x3F800000), jnp.float32)
        idx = ((f - jnp.float32(1.0)) * jnp.float32(_N)).astype(jnp.int32)
        mask_ref[idx] = jnp.int32(1)

    pltpu.sync_copy(mask_ref, out_ref)


def kernel(x):
    del x
    run = pl.kernel(
        _mask_body,
        out_type=jax.ShapeDtypeStruct((_PAD,), jnp.int32),
        mesh=plsc.ScalarSubcoreMesh(axis_name="c", num_cores=1),
        scratch_types=[pltpu.SMEM((_PAD,), jnp.int32)],
        compiler_params=pltpu.CompilerParams(needs_layout_passes=False),
    )
    out = run()
    return out[:_N].astype(jnp.bool_)
